# baseline (speedup 1.0000x reference)
import functools

import jax
import jax.numpy as jnp
from jax import lax
from jax.experimental import pallas as pl
from jax.experimental.pallas import tpu as pltpu
from jax.experimental.pallas import tpu_sc as plsc

EMBED_DIM = 64
MAX_REL_POS = 128
LENGTH_Q = 2048
LENGTH_K = 2048

NUM_CORES = 2
NUM_SUBCORES = 16
NUM_WORKERS = NUM_CORES * NUM_SUBCORES

PAIR_W = 2 * EMBED_DIM
BAND_PAIRS = 2048
BAND_ROWS = 2 * BAND_PAIRS
CHUNK = 128
LANES = 16
ROWS_PER_WORKER = LENGTH_Q // NUM_WORKERS


def _sc_band_kernel(pairs_hbm, idx_hbm, out_hbm, idx_v, rows_v, repack_v,
                    band_sh, sem):
    cid = lax.axis_index("c")
    sid = lax.axis_index("s")

    for half in (0, 1):
        pltpu.sync_copy(
            idx_hbm.at[pl.ds((half * NUM_SUBCORES + sid) * CHUNK, CHUNK)],
            idx_v)
        pltpu.async_copy(pairs_hbm.at[idx_v], rows_v, sem).wait()
        for p in range(CHUNK):
            for k in range(PAIR_W // LANES):
                v = rows_v[p, pl.ds(k * LANES, LANES)]
                repack_v[2 * p + k // 4, pl.ds((k % 4) * LANES, LANES)] = v
        pltpu.sync_copy(
            repack_v,
            band_sh.at[pl.ds((half * NUM_SUBCORES + sid) * 2 * CHUNK,
                             2 * CHUNK)])
    plsc.subcore_barrier()

    wid = sid * NUM_CORES + cid
    copies = []
    for r in range(ROWS_PER_WORKER):
        i = wid * ROWS_PER_WORKER + r
        if r % 2 == 0:
            src = band_sh.at[pl.ds(BAND_ROWS + (LENGTH_Q - 2) - i, LENGTH_K)]
        else:
            src = band_sh.at[pl.ds((LENGTH_Q - 1) - i, LENGTH_K)]
        copies.append(pltpu.async_copy(src, out_hbm.at[i], sem))
    for cp in copies:
        cp.wait()


@functools.partial(
    pl.kernel,
    out_type=jax.ShapeDtypeStruct((LENGTH_Q, LENGTH_K, EMBED_DIM),
                                  jnp.float32),
    mesh=plsc.VectorSubcoreMesh(core_axis_name="c", subcore_axis_name="s"),
    scratch_types=[
        pltpu.VMEM((CHUNK,), jnp.int32),
        pltpu.VMEM((CHUNK, PAIR_W), jnp.float32),
        pltpu.VMEM((2 * CHUNK, EMBED_DIM), jnp.float32),
        pltpu.VMEM_SHARED((2 * BAND_ROWS, EMBED_DIM), jnp.float32),
        pltpu.SemaphoreType.DMA,
    ],
)
def _band_expand_and_replicate(pairs_hbm, idx_hbm, out_hbm, *scratch):
    _sc_band_kernel(pairs_hbm, idx_hbm, out_hbm, *scratch)


def kernel(length_q, length_k, embeddings_table):
    d0 = length_k - length_q
    m = jnp.arange(BAND_ROWS + 2)
    idx = (jnp.clip(m + d0 - (LENGTH_Q - 1), -MAX_REL_POS, MAX_REL_POS)
           + MAX_REL_POS)
    def pair_ids(a, b):
        return jnp.where(a == b, jnp.where(a == 0, 0, 257), a + 1)
    even_ids = pair_ids(idx[0:4096:2], idx[1:4096:2])
    odd_ids = pair_ids(idx[1:4097:2], idx[2:4098:2])
    pair_idx = jnp.concatenate([even_ids, odd_ids]).astype(jnp.int32)
    t0 = jnp.concatenate([embeddings_table[:1], embeddings_table[:1]], axis=1)
    mid = jnp.concatenate([embeddings_table[:-1], embeddings_table[1:]], axis=1)
    t256 = jnp.concatenate([embeddings_table[-1:], embeddings_table[-1:]],
                           axis=1)
    pair_table = jnp.concatenate([t0, mid, t256], axis=0)
    return _band_expand_and_replicate(pair_table, pair_idx)


# speedup vs baseline: 1.8389x; 1.8389x over previous
import functools

import jax
import jax.numpy as jnp
from jax import lax
from jax.experimental import pallas as pl
from jax.experimental.pallas import tpu as pltpu
from jax.experimental.pallas import tpu_sc as plsc

EMBED_DIM = 64
MAX_REL_POS = 128
LENGTH_Q = 2048
LENGTH_K = 2048

NUM_CORES = 2
NUM_SUBCORES = 16
NUM_WORKERS = NUM_CORES * NUM_SUBCORES

N_STAG = 8
MID_OFF = LENGTH_Q - MAX_REL_POS
MID_W = 2 * MAX_REL_POS
SROW = 4112
LO_W = MID_OFF
STRIP_OFF = MID_OFF - 16
STRIP_W = 384
HI_OFF = STRIP_OFF + STRIP_W
HI_W = SROW - HI_OFF

E_SLICE = 2
BLOCK_WORDS = EMBED_DIM * LENGTH_K


def _sc_band_kernel(lo_hbm, strip_hbm, hi_hbm, out_hbm, band_v, sem):
    cid = lax.axis_index("c")
    sid = lax.axis_index("s")
    wid = sid * NUM_CORES + cid

    for phi in range(N_STAG):
        for r in range(E_SLICE):
            e = wid * E_SLICE + r
            row = (phi * E_SLICE + r) * SROW
            pltpu.sync_copy(lo_hbm.at[pl.ds(e * LO_W, LO_W)],
                            band_v.at[pl.ds(row, LO_W)])
            pltpu.sync_copy(
                strip_hbm.at[pl.ds((phi * EMBED_DIM + e) * STRIP_W, STRIP_W)],
                band_v.at[pl.ds(row + STRIP_OFF, STRIP_W)])
            pltpu.sync_copy(hi_hbm.at[pl.ds(e * HI_W, HI_W)],
                            band_v.at[pl.ds(row + HI_OFF, HI_W)])

    def block_body(i, _):
        s = (LENGTH_Q - 1) - i
        phi = jnp.bitwise_and(s, N_STAG - 1)
        sa = pl.multiple_of(s - phi, N_STAG)
        dst_base = i * BLOCK_WORDS + wid * (E_SLICE * LENGTH_K)
        for r in range(E_SLICE):
            pltpu.async_copy(
                band_v.at[pl.ds((phi * E_SLICE + r) * SROW + sa, LENGTH_K)],
                out_hbm.at[pl.ds(dst_base + r * LENGTH_K, LENGTH_K)],
                sem)
        return 0

    lax.fori_loop(0, LENGTH_Q, block_body, 0)
    drain_words = 32768
    n_drains = LENGTH_Q * E_SLICE * LENGTH_K // drain_words

    def drain_body(n, _):
        pltpu.make_async_copy(out_hbm.at[pl.ds(0, drain_words)],
                              band_v.at[pl.ds(0, drain_words)], sem).wait()
        return 0

    lax.fori_loop(0, n_drains, drain_body, 0)


@functools.partial(
    pl.kernel,
    out_type=jax.ShapeDtypeStruct((LENGTH_Q * EMBED_DIM * LENGTH_K,),
                                  jnp.float32),
    mesh=plsc.VectorSubcoreMesh(core_axis_name="c", subcore_axis_name="s"),
    scratch_types=[
        pltpu.VMEM((N_STAG * E_SLICE * SROW,), jnp.float32),
        pltpu.SemaphoreType.DMA,
    ],
)
def _band_expand_and_replicate(lo_hbm, strip_hbm, hi_hbm, out_hbm, *scratch):
    _sc_band_kernel(lo_hbm, strip_hbm, hi_hbm, out_hbm, *scratch)


def kernel(length_q, length_k, embeddings_table):
    table_t = embeddings_table.T
    lo = jnp.broadcast_to(table_t[:, :1], (EMBED_DIM, LO_W)).reshape(-1)
    hi = jnp.broadcast_to(table_t[:, -1:], (EMBED_DIM, HI_W)).reshape(-1)
    strips = []
    for phi in range(N_STAG):
        lo_pad = 16 - phi
        hi_pad = STRIP_W - MID_W - lo_pad
        strips.append(jnp.concatenate(
            [jnp.broadcast_to(table_t[:, :1], (EMBED_DIM, lo_pad)),
             table_t[:, 1:],
             jnp.broadcast_to(table_t[:, -1:], (EMBED_DIM, hi_pad))],
            axis=1))
    strip = jnp.stack(strips).reshape(-1)
    flat = _band_expand_and_replicate(lo, strip, hi)
    return flat.reshape(LENGTH_Q, EMBED_DIM, LENGTH_K).transpose(0, 2, 1)
